# baseline (speedup 1.0000x reference)
import functools

import jax
import jax.numpy as jnp
from jax import lax
from jax.experimental import pallas as pl
from jax.experimental.pallas import tpu as pltpu
from jax.experimental.pallas import tpu_sc as plsc

HIDDEN = 2048
NUM_EXPERTS = 64
TOPK = 8
TB = 1024
NWORKERS = 32
LANES = 16
CHUNK = 256


def _matmul_body(x_ref, wt_ref, b_ref, logits_ref, lt_ref):
    l = jnp.dot(x_ref[...], wt_ref[...],
                preferred_element_type=jnp.float32) + b_ref[...]
    logits_ref[...] = l
    lt_ref[...] = l.T


@jax.jit
def _matmul(x, wt, b2d):
    n = x.shape[0]
    grid = (n // TB,)
    return pl.pallas_call(
        _matmul_body,
        grid=grid,
        in_specs=[
            pl.BlockSpec((TB, HIDDEN), lambda i: (i, 0)),
            pl.BlockSpec((HIDDEN, NUM_EXPERTS), lambda i: (0, 0)),
            pl.BlockSpec((1, NUM_EXPERTS), lambda i: (0, 0)),
        ],
        out_specs=[
            pl.BlockSpec((TB, NUM_EXPERTS), lambda i: (i, 0)),
            pl.BlockSpec((NUM_EXPERTS, TB), lambda i: (0, i)),
        ],
        out_shape=[
            jax.ShapeDtypeStruct((n, NUM_EXPERTS), jnp.float32),
            jax.ShapeDtypeStruct((NUM_EXPERTS, n), jnp.float32),
        ],
        compiler_params=pltpu.CompilerParams(
            dimension_semantics=("arbitrary",),
        ),
    )(x, wt, b2d)


def _group(loff, ooff, lt_v, idx_v, w_v):
    rv = [jnp.full((LANES,), -jnp.inf, jnp.float32)] * TOPK
    ri = [jnp.zeros((LANES,), jnp.int32)] * TOPK
    for e in range(NUM_EXPERTS):
        v = lt_v[e, pl.ds(loff, LANES)]
        iv = jnp.full((LANES,), e, jnp.int32)
        c = [v > rv[j] for j in range(TOPK)]
        nrv = [jnp.where(c[0], v, rv[0])]
        nri = [jnp.where(c[0], iv, ri[0])]
        for j in range(1, TOPK):
            nrv.append(jnp.where(c[j - 1], rv[j - 1],
                                 jnp.where(c[j], v, rv[j])))
            nri.append(jnp.where(c[j - 1], ri[j - 1],
                                 jnp.where(c[j], iv, ri[j])))
        rv, ri = nrv, nri
    es = [jnp.exp(rv[j] - rv[0]) for j in range(TOPK)]
    s = es[0]
    for j in range(1, TOPK):
        s = s + es[j]
    inv = 1.0 / s
    for j in range(TOPK):
        idx_v[j, pl.ds(ooff, LANES)] = ri[j]
        w_v[j, pl.ds(ooff, LANES)] = es[j] * inv


@jax.jit
def _sc_topk(lt):
    n = lt.shape[1]
    per = n // NWORKERS
    nchunks = per // CHUNK
    gpc = CHUNK // LANES
    mesh = plsc.VectorSubcoreMesh(core_axis_name="c", subcore_axis_name="s")

    @functools.partial(
        pl.kernel,
        out_type=[
            jax.ShapeDtypeStruct((TOPK, n), jnp.int32),
            jax.ShapeDtypeStruct((TOPK, n), jnp.float32),
        ],
        mesh=mesh,
        scratch_types=[
            pltpu.VMEM((NUM_EXPERTS, 2 * CHUNK), jnp.float32),
            pltpu.VMEM((TOPK, per), jnp.int32),
            pltpu.VMEM((TOPK, per), jnp.float32),
            pltpu.SemaphoreType.DMA,
        ],
    )
    def k(lt_hbm, idx_hbm, w_hbm, lt_v, idx_v, w_v, sem):
        wid = lax.axis_index("s") * 2 + lax.axis_index("c")
        base = wid * per

        def start(ch, buf):
            pltpu.async_copy(
                lt_hbm.at[:, pl.ds(base + ch * CHUNK, CHUNK)],
                lt_v.at[:, pl.ds(buf * CHUNK, CHUNK)], sem)

        def drain():
            pltpu.make_async_copy(
                lt_hbm.at[:, pl.ds(base, CHUNK)],
                lt_v.at[:, pl.ds(0, CHUNK)], sem).wait()

        start(0, 0)

        @pl.loop(0, nchunks)
        def _(ch):
            @pl.when(ch + 1 < nchunks)
            def _():
                start(ch + 1, (ch + 1) % 2)

            drain()
            buf_off = (ch % 2) * CHUNK

            @pl.loop(0, gpc)
            def _(g):
                _group(buf_off + g * LANES, ch * CHUNK + g * LANES,
                       lt_v, idx_v, w_v)

        pltpu.sync_copy(idx_v, idx_hbm.at[:, pl.ds(base, per)])
        pltpu.sync_copy(w_v, w_hbm.at[:, pl.ds(base, per)])

    return k(lt)


def kernel(hidden_states, W, b):
    B, S, H = hidden_states.shape
    x = hidden_states.reshape(B * S, H)
    logits, lt = _matmul(x, W.T, b.reshape(1, NUM_EXPERTS))
    idx_t, w_t = _sc_topk(lt)
    return (logits.reshape(B, S, NUM_EXPERTS),
            idx_t.T.reshape(B, S, TOPK),
            w_t.T.reshape(B, S, TOPK))


# speedup vs baseline: 1.0063x; 1.0063x over previous
import functools

import jax
import jax.numpy as jnp
from jax import lax
from jax.experimental import pallas as pl
from jax.experimental.pallas import tpu as pltpu
from jax.experimental.pallas import tpu_sc as plsc

HIDDEN = 2048
NUM_EXPERTS = 64
TOPK = 8
TB = 1024
NWORKERS = 32
LANES = 16
CHUNK = 128


def _matmul_body(x_ref, wt_ref, b_ref, logits_ref, lt_ref):
    l = jnp.dot(x_ref[...], wt_ref[...],
                preferred_element_type=jnp.float32) + b_ref[...]
    logits_ref[...] = l
    lt_ref[...] = l.T


@jax.jit
def _matmul(x, wt, b2d):
    n = x.shape[0]
    grid = (n // TB,)
    return pl.pallas_call(
        _matmul_body,
        grid=grid,
        in_specs=[
            pl.BlockSpec((TB, HIDDEN), lambda i: (i, 0)),
            pl.BlockSpec((HIDDEN, NUM_EXPERTS), lambda i: (0, 0)),
            pl.BlockSpec((1, NUM_EXPERTS), lambda i: (0, 0)),
        ],
        out_specs=[
            pl.BlockSpec((TB, NUM_EXPERTS), lambda i: (i, 0)),
            pl.BlockSpec((NUM_EXPERTS, TB), lambda i: (0, i)),
        ],
        out_shape=[
            jax.ShapeDtypeStruct((n, NUM_EXPERTS), jnp.float32),
            jax.ShapeDtypeStruct((NUM_EXPERTS, n), jnp.float32),
        ],
        compiler_params=pltpu.CompilerParams(
            dimension_semantics=("arbitrary",),
        ),
    )(x, wt, b2d)


def _group(loff, ooff, lt_v, idx_v, w_v):
    rv = [jnp.full((LANES,), -jnp.inf, jnp.float32)] * TOPK
    ri = [jnp.zeros((LANES,), jnp.int32)] * TOPK
    for e in range(NUM_EXPERTS):
        v = lt_v[e, pl.ds(loff, LANES)]
        iv = jnp.full((LANES,), e, jnp.int32)
        c = [v > rv[j] for j in range(TOPK)]
        nrv = [jnp.where(c[0], v, rv[0])]
        nri = [jnp.where(c[0], iv, ri[0])]
        for j in range(1, TOPK):
            nrv.append(jnp.where(c[j - 1], rv[j - 1],
                                 jnp.where(c[j], v, rv[j])))
            nri.append(jnp.where(c[j - 1], ri[j - 1],
                                 jnp.where(c[j], iv, ri[j])))
        rv, ri = nrv, nri
    es = [jnp.exp(rv[j] - rv[0]) for j in range(TOPK)]
    s = es[0]
    for j in range(1, TOPK):
        s = s + es[j]
    inv = 1.0 / s
    for j in range(TOPK):
        idx_v[j, pl.ds(ooff, LANES)] = ri[j]
        w_v[j, pl.ds(ooff, LANES)] = es[j] * inv


@jax.jit
def _sc_topk(lt):
    n = lt.shape[1]
    per = n // NWORKERS
    nchunks = per // CHUNK
    gpc = CHUNK // LANES
    mesh = plsc.VectorSubcoreMesh(core_axis_name="c", subcore_axis_name="s")

    @functools.partial(
        pl.kernel,
        out_type=[
            jax.ShapeDtypeStruct((TOPK, n), jnp.int32),
            jax.ShapeDtypeStruct((TOPK, n), jnp.float32),
        ],
        mesh=mesh,
        scratch_types=[
            pltpu.VMEM((NUM_EXPERTS, 2 * CHUNK), jnp.float32),
            pltpu.VMEM((TOPK, per), jnp.int32),
            pltpu.VMEM((TOPK, per), jnp.float32),
            pltpu.SemaphoreType.DMA,
        ],
    )
    def k(lt_hbm, idx_hbm, w_hbm, lt_v, idx_v, w_v, sem):
        wid = lax.axis_index("s") * 2 + lax.axis_index("c")
        base = wid * per

        def start(ch, buf):
            pltpu.async_copy(
                lt_hbm.at[:, pl.ds(base + ch * CHUNK, CHUNK)],
                lt_v.at[:, pl.ds(buf * CHUNK, CHUNK)], sem)

        def drain():
            pltpu.make_async_copy(
                lt_hbm.at[:, pl.ds(base, CHUNK)],
                lt_v.at[:, pl.ds(0, CHUNK)], sem).wait()

        start(0, 0)

        @pl.loop(0, nchunks)
        def _(ch):
            @pl.when(ch + 1 < nchunks)
            def _():
                start(ch + 1, (ch + 1) % 2)

            drain()
            buf_off = (ch % 2) * CHUNK

            @pl.loop(0, gpc)
            def _(g):
                _group(buf_off + g * LANES, ch * CHUNK + g * LANES,
                       lt_v, idx_v, w_v)

        pltpu.sync_copy(idx_v, idx_hbm.at[:, pl.ds(base, per)])
        pltpu.sync_copy(w_v, w_hbm.at[:, pl.ds(base, per)])

    return k(lt)


def kernel(hidden_states, W, b):
    B, S, H = hidden_states.shape
    x = hidden_states.reshape(B * S, H)
    logits, lt = _matmul(x, W.T, b.reshape(1, NUM_EXPERTS))
    idx_t, w_t = _sc_topk(lt)
    return (logits.reshape(B, S, NUM_EXPERTS),
            idx_t.T.reshape(B, S, TOPK),
            w_t.T.reshape(B, S, TOPK))


# speedup vs baseline: 1.0082x; 1.0020x over previous
import functools

import jax
import jax.numpy as jnp
from jax import lax
from jax.experimental import pallas as pl
from jax.experimental.pallas import tpu as pltpu
from jax.experimental.pallas import tpu_sc as plsc

HIDDEN = 2048
NUM_EXPERTS = 64
TOPK = 8
TB = 1024
NWORKERS = 32
LANES = 16
CHUNK = 128


def _matmul_body(x_ref, wt_ref, b_ref, logits_ref, lt_ref):
    l = jnp.dot(x_ref[...], wt_ref[...],
                preferred_element_type=jnp.float32) + b_ref[...]
    logits_ref[...] = l
    lt_ref[...] = l.T


@jax.jit
def _matmul(x, wt, b2d):
    n = x.shape[0]
    grid = (n // TB,)
    return pl.pallas_call(
        _matmul_body,
        grid=grid,
        in_specs=[
            pl.BlockSpec((TB, HIDDEN), lambda i: (i, 0)),
            pl.BlockSpec((HIDDEN, NUM_EXPERTS), lambda i: (0, 0)),
            pl.BlockSpec((1, NUM_EXPERTS), lambda i: (0, 0)),
        ],
        out_specs=[
            pl.BlockSpec((TB, NUM_EXPERTS), lambda i: (i, 0)),
            pl.BlockSpec((NUM_EXPERTS, TB), lambda i: (0, i)),
        ],
        out_shape=[
            jax.ShapeDtypeStruct((n, NUM_EXPERTS), jnp.float32),
            jax.ShapeDtypeStruct((NUM_EXPERTS, n), jnp.float32),
        ],
        compiler_params=pltpu.CompilerParams(
            dimension_semantics=("parallel",),
        ),
    )(x, wt, b2d)


def _group(loff, ooff, lt_v, idx_v, w_v):
    rv = [jnp.full((LANES,), -jnp.inf, jnp.float32)] * TOPK
    ri = [jnp.zeros((LANES,), jnp.int32)] * TOPK
    for e in range(NUM_EXPERTS):
        v = lt_v[e, pl.ds(loff, LANES)]
        iv = jnp.full((LANES,), e, jnp.int32)
        c = [v > rv[j] for j in range(TOPK)]
        nrv = [jnp.where(c[0], v, rv[0])]
        nri = [jnp.where(c[0], iv, ri[0])]
        for j in range(1, TOPK):
            nrv.append(jnp.where(c[j - 1], rv[j - 1],
                                 jnp.where(c[j], v, rv[j])))
            nri.append(jnp.where(c[j - 1], ri[j - 1],
                                 jnp.where(c[j], iv, ri[j])))
        rv, ri = nrv, nri
    es = [jnp.exp(rv[j] - rv[0]) for j in range(TOPK)]
    s = es[0]
    for j in range(1, TOPK):
        s = s + es[j]
    inv = 1.0 / s
    for j in range(TOPK):
        idx_v[j, pl.ds(ooff, LANES)] = ri[j]
        w_v[j, pl.ds(ooff, LANES)] = es[j] * inv


@jax.jit
def _sc_topk(lt):
    n = lt.shape[1]
    per = n // NWORKERS
    nchunks = per // CHUNK
    gpc = CHUNK // LANES
    mesh = plsc.VectorSubcoreMesh(core_axis_name="c", subcore_axis_name="s")

    @functools.partial(
        pl.kernel,
        out_type=[
            jax.ShapeDtypeStruct((TOPK, n), jnp.int32),
            jax.ShapeDtypeStruct((TOPK, n), jnp.float32),
        ],
        mesh=mesh,
        scratch_types=[
            pltpu.VMEM((NUM_EXPERTS, 2 * CHUNK), jnp.float32),
            pltpu.VMEM((TOPK, per), jnp.int32),
            pltpu.VMEM((TOPK, per), jnp.float32),
            pltpu.SemaphoreType.DMA,
        ],
    )
    def k(lt_hbm, idx_hbm, w_hbm, lt_v, idx_v, w_v, sem):
        wid = lax.axis_index("s") * 2 + lax.axis_index("c")
        base = wid * per

        def start(ch, buf):
            pltpu.async_copy(
                lt_hbm.at[:, pl.ds(base + ch * CHUNK, CHUNK)],
                lt_v.at[:, pl.ds(buf * CHUNK, CHUNK)], sem)

        def drain():
            pltpu.make_async_copy(
                lt_hbm.at[:, pl.ds(base, CHUNK)],
                lt_v.at[:, pl.ds(0, CHUNK)], sem).wait()

        start(0, 0)

        @pl.loop(0, nchunks)
        def _(ch):
            @pl.when(ch + 1 < nchunks)
            def _():
                start(ch + 1, (ch + 1) % 2)

            drain()
            buf_off = (ch % 2) * CHUNK

            @pl.loop(0, gpc)
            def _(g):
                _group(buf_off + g * LANES, ch * CHUNK + g * LANES,
                       lt_v, idx_v, w_v)

        pltpu.sync_copy(idx_v, idx_hbm.at[:, pl.ds(base, per)])
        pltpu.sync_copy(w_v, w_hbm.at[:, pl.ds(base, per)])

    return k(lt)


def kernel(hidden_states, W, b):
    B, S, H = hidden_states.shape
    x = hidden_states.reshape(B * S, H)
    logits, lt = _matmul(x, W.T, b.reshape(1, NUM_EXPERTS))
    idx_t, w_t = _sc_topk(lt)
    return (logits.reshape(B, S, NUM_EXPERTS),
            idx_t.T.reshape(B, S, TOPK),
            w_t.T.reshape(B, S, TOPK))
